# Initial kernel scaffold; baseline (speedup 1.0000x reference)
import functools

import jax
import jax.numpy as jnp
from jax import lax
from jax.experimental import pallas as pl
from jax.experimental.pallas import tpu as pltpu
from jax.experimental.pallas import tpu_sc as plsc

V = 1000000
D = 64
R = 8
SCALING = 2.0

NC, NS = 2, 16
NW = NC * NS
TOK = 1024 * 200
PW = TOK // NW
CH = 128
NCHUNK = PW // CH


def _sc_body(x_ref, w_ref, a_ref, b_ref, out_ref,
             idx_v, wrows_v, arows_v, bt_v, wsem, asem):
    cid = lax.axis_index("c")
    sid = lax.axis_index("s")
    wid = sid * NC + cid
    base_row = wid * NCHUNK

    pltpu.sync_copy(x_ref.at[pl.ds(base_row, NCHUNK)], idx_v)
    pltpu.sync_copy(b_ref, bt_v)

    bts = [[bt_v[r, pl.ds(k * 16, 16)] for k in range(D // 16)]
           for r in range(R)]

    def chunk_body(j, carry):
        cw = pltpu.async_copy(w_ref.at[idx_v.at[j]], wrows_v, wsem)
        ca = pltpu.async_copy(a_ref.at[idx_v.at[j]], arows_v, asem)
        cw.wait()
        ca.wait()

        def tok_body(t, tc):
            accs = [wrows_v[t, pl.ds(k * 16, 16)] for k in range(D // 16)]
            for r in range(R):
                ar = plsc.load_gather(
                    arows_v,
                    [jnp.full((16,), t, jnp.int32),
                     jnp.full((16,), r, jnp.int32)])
                for k in range(D // 16):
                    accs[k] = accs[k] + ar * bts[r][k]
            for k in range(D // 16):
                wrows_v[t, pl.ds(k * 16, 16)] = accs[k]
            return tc

        lax.fori_loop(0, CH, tok_body, 0)
        tok_base = (base_row + j) * CH
        pltpu.sync_copy(wrows_v, out_ref.at[pl.ds(tok_base, CH)])
        return carry

    lax.fori_loop(0, NCHUNK, chunk_body, 0)


_sc_lora_embed = functools.partial(
    pl.kernel,
    out_type=jax.ShapeDtypeStruct((TOK, D), jnp.float32),
    mesh=plsc.VectorSubcoreMesh(core_axis_name="c", subcore_axis_name="s"),
    scratch_types=[
        pltpu.VMEM((NCHUNK, CH), jnp.int32),
        pltpu.VMEM((CH, D), jnp.float32),
        pltpu.VMEM((CH, R), jnp.float32),
        pltpu.VMEM((R, D), jnp.float32),
        pltpu.SemaphoreType.DMA,
        pltpu.SemaphoreType.DMA,
    ],
)(_sc_body)


@jax.jit
def kernel(x, weight, lora_A, lora_B):
    B, L = x.shape
    x2d = x.reshape(TOK // CH, CH)
    a_t = lora_A.T
    bt = (SCALING * lora_B).T
    out = _sc_lora_embed(x2d, weight, a_t, bt)
    return out.reshape(B, L, D)


# speedup vs baseline: 2.6813x; 2.6813x over previous
import functools

import jax
import jax.numpy as jnp
from jax import lax
from jax.experimental import pallas as pl
from jax.experimental.pallas import tpu as pltpu
from jax.experimental.pallas import tpu_sc as plsc

V = 1000000
D = 64
R = 8
SCALING = 2.0

NC, NS = 2, 16
NW = NC * NS
TOK = 1024 * 200
PW = TOK // NW
CH = 128
NCHUNK = PW // CH


def _sc_body(x_ref, w_ref, a_ref, b_ref, out_ref,
             idx_v, wrows_v, arows_v, bt_v, wsem, asem):
    cid = lax.axis_index("c")
    sid = lax.axis_index("s")
    wid = sid * NC + cid
    base_row = wid * NCHUNK

    pltpu.sync_copy(x_ref.at[wid], idx_v)
    pltpu.sync_copy(b_ref, bt_v)

    bts = [[bt_v[r, pl.ds(k * 16, 16)] for k in range(D // 16)]
           for r in range(R)]

    def chunk_body(j, carry):
        cw = pltpu.async_copy(w_ref.at[idx_v.at[j]], wrows_v, wsem)
        ca = pltpu.async_copy(a_ref.at[idx_v.at[j]], arows_v, asem)
        cw.wait()
        ca.wait()

        def tok_body(t, tc):
            accs = [wrows_v[t, pl.ds(k * 16, 16)] for k in range(D // 16)]
            for r in range(R):
                ar = plsc.load_gather(
                    arows_v,
                    [jnp.full((16,), t, jnp.int32),
                     jnp.full((16,), r, jnp.int32)])
                for k in range(D // 16):
                    accs[k] = accs[k] + ar * bts[r][k]
            for k in range(D // 16):
                wrows_v[t, pl.ds(k * 16, 16)] = accs[k]
            return tc

        lax.fori_loop(0, CH, tok_body, 0)
        tok_base = (base_row + j) * CH
        pltpu.sync_copy(wrows_v, out_ref.at[pl.ds(tok_base, CH)])
        return carry

    lax.fori_loop(0, NCHUNK, chunk_body, 0)


@functools.cache
def _sc_lora_embed():
    return functools.partial(
        pl.kernel,
        out_type=jax.ShapeDtypeStruct((TOK, D), jnp.float32),
        mesh=plsc.VectorSubcoreMesh(core_axis_name="c", subcore_axis_name="s"),
        compiler_params=pltpu.CompilerParams(
            use_tc_tiling_on_sc=False, needs_layout_passes=False),
        scratch_types=[
            pltpu.VMEM((NCHUNK, CH), jnp.int32),
            pltpu.VMEM((CH, D), jnp.float32),
            pltpu.VMEM((CH, R), jnp.float32),
            pltpu.VMEM((R, D), jnp.float32),
            pltpu.SemaphoreType.DMA,
            pltpu.SemaphoreType.DMA,
        ],
    )(_sc_body)


@jax.jit
def kernel(x, weight, lora_A, lora_B):
    B, L = x.shape
    x3d = x.reshape(NW, NCHUNK, CH)
    a_t = lora_A.T
    bt = (SCALING * lora_B).T
    out = _sc_lora_embed()(x3d, weight, a_t, bt)
    return out.reshape(B, L, D)


# speedup vs baseline: 2.8043x; 1.0459x over previous
import functools

import jax
import jax.numpy as jnp
from jax import lax
from jax.experimental import pallas as pl
from jax.experimental.pallas import tpu as pltpu
from jax.experimental.pallas import tpu_sc as plsc

V = 1000000
D = 64
R = 8
SCALING = 2.0

NC, NS = 2, 16
NW = NC * NS
TOK = 1024 * 200
PW = TOK // NW
CH = 128
NCHUNK = PW // CH


GC = 5
TG = GC * CH
NG = NCHUNK // GC


def _sc_body(x_ref, w_ref, a_ref, b_ref, out_ref,
             idx_v, wrows_v, arows_v, bt_v, wsem, asem, osem0, osem1):
    cid = lax.axis_index("c")
    sid = lax.axis_index("s")
    wid = sid * NC + cid
    tok0 = wid * PW

    pltpu.sync_copy(x_ref.at[wid], idx_v)
    pltpu.sync_copy(b_ref, bt_v)

    bts = [[bt_v[r, pl.ds(k * 16, 16)] for k in range(D // 16)]
           for r in range(R)]
    osems = (osem0, osem1)

    def gathers(g, slot):
        cps = []
        for c in range(GC):
            j = g * GC + c
            cps.append(pltpu.make_async_copy(
                w_ref.at[idx_v.at[j]],
                wrows_v.at[slot, pl.ds(c * CH, CH)], wsem))
            cps.append(pltpu.make_async_copy(
                a_ref.at[idx_v.at[j]],
                arows_v.at[slot, pl.ds(c * CH, CH)], asem))
        return cps

    def outcopy(g, slot):
        return pltpu.make_async_copy(
            wrows_v.at[slot], out_ref.at[pl.ds(tok0 + g * TG, TG)],
            osems[slot])

    def compute(slot):
        def tok_body(t, tc):
            accs = [wrows_v[slot, t, pl.ds(k * 16, 16)]
                    for k in range(D // 16)]
            for r in range(R):
                ar = plsc.load_gather(
                    arows_v.at[slot],
                    [jnp.full((16,), t, jnp.int32),
                     jnp.full((16,), r, jnp.int32)])
                for k in range(D // 16):
                    accs[k] = accs[k] + ar * bts[r][k]
            for k in range(D // 16):
                wrows_v[slot, t, pl.ds(k * 16, 16)] = accs[k]
            return tc
        lax.fori_loop(0, TG, tok_body, 0, unroll=2)

    for cp in gathers(0, 0):
        cp.start()
    for g in range(NG):
        slot = g & 1
        if g + 1 < NG:
            if g >= 1:
                outcopy(g - 1, 1 - slot).wait()
            for cp in gathers(g + 1, 1 - slot):
                cp.start()
        for cp in gathers(g, slot):
            cp.wait()
        compute(slot)
        outcopy(g, slot).start()
    outcopy(NG - 2, 0).wait()
    outcopy(NG - 1, 1).wait()


@functools.cache
def _sc_lora_embed():
    return functools.partial(
        pl.kernel,
        out_type=jax.ShapeDtypeStruct((TOK, D), jnp.float32),
        mesh=plsc.VectorSubcoreMesh(core_axis_name="c", subcore_axis_name="s"),
        compiler_params=pltpu.CompilerParams(
            use_tc_tiling_on_sc=False, needs_layout_passes=False),
        scratch_types=[
            pltpu.VMEM((NCHUNK, CH), jnp.int32),
            pltpu.VMEM((2, TG, D), jnp.float32),
            pltpu.VMEM((2, TG, R), jnp.float32),
            pltpu.VMEM((R, D), jnp.float32),
            pltpu.SemaphoreType.DMA,
            pltpu.SemaphoreType.DMA,
            pltpu.SemaphoreType.DMA,
            pltpu.SemaphoreType.DMA,
        ],
    )(_sc_body)


@jax.jit
def kernel(x, weight, lora_A, lora_B):
    B, L = x.shape
    x3d = x.reshape(NW, NCHUNK, CH)
    a_t = lora_A.T
    bt = (SCALING * lora_B).T
    out = _sc_lora_embed()(x3d, weight, a_t, bt)
    return out.reshape(B, L, D)
